# baseline (speedup 1.0000x reference)
import functools

import jax
import jax.numpy as jnp
from jax import lax
from jax.experimental import pallas as pl
from jax.experimental.pallas import tpu as pltpu
from jax.experimental.pallas import tpu_sc as plsc

_VOCAB = 100000
_D = 64
_B = 1024
_CTX = 20

_NC, _NS = 2, 16
_NW = _NC * _NS
_BPW = _B // _NW
_IPW = _BPW * _CTX
_ICHUNK = 128
_NCHUNK = _IPW // _ICHUNK
_LANES = 16


@functools.lru_cache(maxsize=None)
def _make_pool():
    mesh = plsc.VectorSubcoreMesh(
        core_axis_name="c", subcore_axis_name="s",
        num_cores=_NC, num_subcores=_NS,
    )

    @functools.partial(
        pl.kernel,
        mesh=mesh,
        out_type=jax.ShapeDtypeStruct((_B, _D), jnp.float32),
        scratch_types=[
            pltpu.VMEM((_IPW,), jnp.int32),
            pltpu.VMEM((_IPW,), jnp.int32),
            pltpu.VMEM((_IPW, 2 * _D), jnp.float32),
            pltpu.VMEM((_BPW, _D), jnp.float32),
            pltpu.SemaphoreType.DMA,
        ],
        compiler_params=pltpu.CompilerParams(needs_layout_passes=False),
    )
    def _pool(ctx_hbm, table_hbm, out_hbm, idx_v, pair_v, rows_v, pooled_v, sem):
        wid = lax.axis_index("s") * _NC + lax.axis_index("c")
        pltpu.sync_copy(ctx_hbm.at[pl.ds(wid * _IPW, _IPW)], idx_v)

        def mk_pairs(r, carry):
            sl = pl.ds(r * _LANES, _LANES)
            iv = idx_v[sl]
            pair_v[sl] = jnp.where(iv >= _FOLD, iv - _FOLD, iv)
            return carry

        lax.fori_loop(0, _IPW // _LANES, mk_pairs, 0)
        copies = [
            pltpu.async_copy(
                table_hbm.at[pair_v.at[pl.ds(j * _ICHUNK, _ICHUNK)]],
                rows_v.at[pl.ds(j * _ICHUNK, _ICHUNK)],
                sem,
            )
            for j in range(_NCHUNK)
        ]
        for cp in copies:
            cp.wait()

        def body(b, carry):
            accs = [jnp.zeros((_LANES,), jnp.float32)] * (_D // _LANES)
            for c in range(_CTX):
                r = b * _CTX + c
                splat = plsc.load_gather(idx_v, [jnp.full((_LANES,), r, jnp.int32)])
                odd = splat >= _FOLD
                for d in range(_D // _LANES):
                    lo = rows_v[r, pl.ds(d * _LANES, _LANES)]
                    hi = rows_v[r, pl.ds(_D + d * _LANES, _LANES)]
                    accs[d] = accs[d] + jnp.where(odd, hi, lo)
            for d in range(_D // _LANES):
                pooled_v[b, pl.ds(d * _LANES, _LANES)] = accs[d] * (1.0 / _CTX)
            return carry

        lax.fori_loop(0, _BPW, body, 0)
        pltpu.sync_copy(pooled_v, out_hbm.at[pl.ds(wid * _BPW, _BPW)])

    return _pool


_FOLD = 51200
_TVB = 12800
_TNB = _FOLD // _TVB


def _pair_kernel(a_ref, b_ref, o_ref):
    o_ref[:, 0:_D] = a_ref[...].T
    o_ref[:, _D:2 * _D] = b_ref[...].T


def _make_pairs(tableT):
    return pl.pallas_call(
        _pair_kernel,
        grid=(_TNB,),
        in_specs=[
            pl.BlockSpec((_D, _TVB), lambda v: (0, v)),
            pl.BlockSpec(
                (_D, _TVB),
                lambda v: (0, jnp.minimum(v + _TNB, (_VOCAB - 1) // _TVB)),
            ),
        ],
        out_specs=pl.BlockSpec((_TVB, 2 * _D), lambda v: (v, 0)),
        out_shape=jax.ShapeDtypeStruct((_FOLD, 2 * _D), jnp.float32),
        compiler_params=pltpu.CompilerParams(
            dimension_semantics=("parallel",),
        ),
    )(tableT, tableT)


_VB = 4096
_NVB = (_VOCAB + _VB - 1) // _VB


def _project_kernel(w_ref, p_ref, b_ref, o_ref):
    o_ref[...] = lax.dot_general(
        w_ref[...], p_ref[...],
        dimension_numbers=(((0,), (1,)), ((), ())),
        preferred_element_type=jnp.float32,
    ) + b_ref[...].T


def _project(WT, pooled, b2):
    return pl.pallas_call(
        _project_kernel,
        grid=(_NVB,),
        in_specs=[
            pl.BlockSpec((_D, _VB), lambda v: (0, v)),
            pl.BlockSpec((_B, _D), lambda v: (0, 0)),
            pl.BlockSpec((1, _VB), lambda v: (0, v)),
        ],
        out_specs=pl.BlockSpec((_VB, _B), lambda v: (v, 0)),
        out_shape=jax.ShapeDtypeStruct((_VOCAB, _B), jnp.float32),
        compiler_params=pltpu.CompilerParams(
            dimension_semantics=("parallel",),
        ),
    )(WT, pooled, b2)


def kernel(context, emb_table, W, b):
    ctx = context.astype(jnp.int32).reshape(_B * _CTX)
    emb2 = _make_pairs(emb_table.T)
    pooled = _make_pool()(ctx, emb2)
    outT = _project(W.T, pooled, b.reshape(1, _VOCAB))
    return outT.T


# speedup vs baseline: 1.0007x; 1.0007x over previous
import functools

import jax
import jax.numpy as jnp
from jax import lax
from jax.experimental import pallas as pl
from jax.experimental.pallas import tpu as pltpu
from jax.experimental.pallas import tpu_sc as plsc

_VOCAB = 100000
_D = 64
_B = 1024
_CTX = 20

_NC, _NS = 2, 16
_NW = _NC * _NS
_BPW = _B // _NW
_IPW = _BPW * _CTX
_ICHUNK = 128
_NCHUNK = _IPW // _ICHUNK
_LANES = 16


@functools.lru_cache(maxsize=None)
def _make_pool():
    mesh = plsc.VectorSubcoreMesh(
        core_axis_name="c", subcore_axis_name="s",
        num_cores=_NC, num_subcores=_NS,
    )

    @functools.partial(
        pl.kernel,
        mesh=mesh,
        out_type=jax.ShapeDtypeStruct((_B, _D), jnp.float32),
        scratch_types=[
            pltpu.VMEM((_IPW,), jnp.int32),
            pltpu.VMEM((_IPW,), jnp.int32),
            pltpu.VMEM((_IPW, 2 * _D), jnp.float32),
            pltpu.VMEM((_BPW, _D), jnp.float32),
            pltpu.SemaphoreType.DMA,
        ],
        compiler_params=pltpu.CompilerParams(needs_layout_passes=False),
    )
    def _pool(ctx_hbm, table_hbm, out_hbm, idx_v, pair_v, rows_v, pooled_v, sem):
        wid = lax.axis_index("s") * _NC + lax.axis_index("c")
        pltpu.sync_copy(ctx_hbm.at[pl.ds(wid * _IPW, _IPW)], idx_v)

        def mk_pairs(r, carry):
            sl = pl.ds(r * _LANES, _LANES)
            iv = idx_v[sl]
            pair_v[sl] = jnp.where(iv >= _FOLD, iv - _FOLD, iv)
            return carry

        lax.fori_loop(0, _IPW // _LANES, mk_pairs, 0)
        copies = [
            pltpu.async_copy(
                table_hbm.at[pair_v.at[pl.ds(j * _ICHUNK, _ICHUNK)]],
                rows_v.at[pl.ds(j * _ICHUNK, _ICHUNK)],
                sem,
            )
            for j in range(_NCHUNK)
        ]
        for cp in copies:
            cp.wait()

        def body(b, carry):
            accs = [jnp.zeros((_LANES,), jnp.float32)] * (_D // _LANES)
            for c in range(_CTX):
                r = b * _CTX + c
                splat = plsc.load_gather(idx_v, [jnp.full((_LANES,), r, jnp.int32)])
                odd = splat >= _FOLD
                for d in range(_D // _LANES):
                    lo = rows_v[r, pl.ds(d * _LANES, _LANES)]
                    hi = rows_v[r, pl.ds(_D + d * _LANES, _LANES)]
                    accs[d] = accs[d] + jnp.where(odd, hi, lo)
            for d in range(_D // _LANES):
                pooled_v[b, pl.ds(d * _LANES, _LANES)] = accs[d] * (1.0 / _CTX)
            return carry

        lax.fori_loop(0, _BPW, body, 0)
        pltpu.sync_copy(pooled_v, out_hbm.at[pl.ds(wid * _BPW, _BPW)])

    return _pool


_FOLD = 51200
_TVB = 6400
_TNB = _FOLD // _TVB


def _pair_kernel(a_ref, b_ref, o_ref):
    o_ref[:, 0:_D] = a_ref[...].T
    o_ref[:, _D:2 * _D] = b_ref[...].T


def _make_pairs(tableT):
    return pl.pallas_call(
        _pair_kernel,
        grid=(_TNB,),
        in_specs=[
            pl.BlockSpec((_D, _TVB), lambda v: (0, v)),
            pl.BlockSpec(
                (_D, _TVB),
                lambda v: (0, jnp.minimum(v + _TNB, (_VOCAB - 1) // _TVB)),
            ),
        ],
        out_specs=pl.BlockSpec((_TVB, 2 * _D), lambda v: (v, 0)),
        out_shape=jax.ShapeDtypeStruct((_FOLD, 2 * _D), jnp.float32),
        compiler_params=pltpu.CompilerParams(
            dimension_semantics=("parallel",),
        ),
    )(tableT, tableT)


_VB = 4096
_NVB = (_VOCAB + _VB - 1) // _VB


def _project_kernel(w_ref, p_ref, b_ref, o_ref):
    o_ref[...] = lax.dot_general(
        w_ref[...], p_ref[...],
        dimension_numbers=(((0,), (1,)), ((), ())),
        preferred_element_type=jnp.float32,
    ) + b_ref[...].T


def _project(WT, pooled, b2):
    return pl.pallas_call(
        _project_kernel,
        grid=(_NVB,),
        in_specs=[
            pl.BlockSpec((_D, _VB), lambda v: (0, v)),
            pl.BlockSpec((_B, _D), lambda v: (0, 0)),
            pl.BlockSpec((1, _VB), lambda v: (0, v)),
        ],
        out_specs=pl.BlockSpec((_VB, _B), lambda v: (v, 0)),
        out_shape=jax.ShapeDtypeStruct((_VOCAB, _B), jnp.float32),
        compiler_params=pltpu.CompilerParams(
            dimension_semantics=("parallel",),
        ),
    )(WT, pooled, b2)


def kernel(context, emb_table, W, b):
    ctx = context.astype(jnp.int32).reshape(_B * _CTX)
    emb2 = _make_pairs(emb_table.T)
    pooled = _make_pool()(ctx, emb2)
    outT = _project(W.T, pooled, b.reshape(1, _VOCAB))
    return outT.T
